# Initial kernel scaffold; baseline (speedup 1.0000x reference)
import jax
import jax.numpy as jnp
from jax.experimental import pallas as pl


def kernel(x, edge_index, W1, b1, W2, b2, Wc, bc):
    raise NotImplementedError("write your pallas kernel here")


# speedup vs baseline: 18.6450x; 18.6450x over previous
import jax
import jax.numpy as jnp
from jax import lax
from jax.experimental import pallas as pl
from jax.experimental.pallas import tpu as pltpu
from jax.experimental.pallas import tpu_sc as plsc

N = 50000
E = 800000
IN_CH = 128
HID = 64
F = 16
NQ = HID // F

NPAD = 50176
EPAD = 819200
EROWS = EPAD // 128
CH_DEG = 200
CH_AGG = 400
CH_ST = 200
TSLICE = NPAD // 16
NBUF = 4

_MESH = plsc.VectorSubcoreMesh(core_axis_name="c", subcore_axis_name="s")


def _deg_body(dstR, hist_out, out0, out1, idx, hist, rbuf, pbuf):
    c = lax.axis_index("c")
    s = lax.axis_index("s")
    w = c * 16 + s

    def zb(i, t):
        hist[pl.ds(i * 16, 16)] = jnp.zeros((16,), jnp.float32)
        return t

    lax.fori_loop(0, NPAD // 16, zb, 0)
    pltpu.sync_copy(dstR.at[pl.ds(w * CH_DEG, CH_DEG)], idx)

    ones16 = jnp.ones((16,), jnp.float32)

    def loop(j, t):
        def inner(k, u):
            ids = idx[j, pl.ds(k * 16, 16)]
            plsc.addupdate_scatter(hist, [ids], ones16)
            return u

        lax.fori_loop(0, 8, inner, 0)
        return t

    lax.fori_loop(0, CH_DEG, loop, 0)

    pltpu.sync_copy(hist, hist_out.at[w])
    plsc.subcore_barrier()

    r0 = s * TSLICE

    def zr(i, t):
        rbuf[pl.ds(i * 16, 16)] = jnp.zeros((16,), jnp.float32)
        return t

    lax.fori_loop(0, TSLICE // 16, zr, 0)
    for t in range(16):
        pltpu.sync_copy(hist_out.at[c * 16 + t, pl.ds(r0, TSLICE)], pbuf)

        def racc(i, u):
            rbuf[pl.ds(i * 16, 16)] = rbuf[pl.ds(i * 16, 16)] + pbuf[pl.ds(i * 16, 16)]
            return u

        lax.fori_loop(0, TSLICE // 16, racc, 0)
    for cid, oref in ((0, out0), (1, out1)):
        @pl.when(c == cid)
        def _wb(oref=oref):
            pltpu.sync_copy(rbuf, oref.at[pl.ds(r0, TSLICE)])


_deg_call = pl.kernel(
    _deg_body,
    out_type=[pltpu.HBM((32, NPAD), jnp.float32),
              pltpu.HBM((NPAD,), jnp.float32),
              pltpu.HBM((NPAD,), jnp.float32)],
    mesh=_MESH,
    scratch_types=(
        [pltpu.VMEM((CH_DEG, 128), jnp.int32),
         pltpu.VMEM((NPAD,), jnp.float32),
         pltpu.VMEM((TSLICE,), jnp.float32),
         pltpu.VMEM((TSLICE,), jnp.float32)]
    ),
    compiler_params=pltpu.CompilerParams(use_tc_tiling_on_sc=False,
                                         needs_layout_passes=False),
)


def _agg_body(srcR, dstR, g0, g1, g2, g3, o0, o1, o2, o3, sidx, didx, rows,
              acc, gs0, gs1, gs2, gs3, as0, as1, as2, as3):
    c = lax.axis_index("c")
    s = lax.axis_index("s")
    gsems = (gs0, gs1, gs2, gs3)
    ssems = (as0, as1, as2, as3)
    grefs = (g0, g1, g2, g3)
    orefs = (o0, o1, o2, o3)
    r0 = s * TSLICE

    for cid in (0, 1):
        @pl.when(c == cid)
        def _core(cid=cid):
            for p in range(NQ // 2):
                q = (NQ // 2) * cid + p
                gq = grefs[q]
                oq = orefs[q]
                for k in range(7):
                    pltpu.sync_copy(gq.at[pl.ds(r0 + k * 448, 448)],
                                    rows.at[pl.ds(0, 448)])
                    pltpu.sync_copy(rows.at[pl.ds(0, 448)],
                                    acc.at[pl.ds(r0 + k * 448, 448)])
                plsc.subcore_barrier()

                for st in range(CH_AGG // CH_ST):
                    e0 = s * CH_AGG + st * CH_ST
                    pltpu.sync_copy(srcR.at[pl.ds(e0, CH_ST)], sidx)
                    pltpu.sync_copy(dstR.at[pl.ds(e0, CH_ST)], didx)

                    for b in range(NBUF):
                        pltpu.async_copy(gq.at[sidx.at[b]],
                                         rows.at[pl.ds(b * 128, 128)],
                                         gsems[b])

                    def loop(o, t, gq=gq):
                        for b in range(NBUF):
                            j = o * NBUF + b
                            buf = rows.at[pl.ds(b * 128, 128)]
                            pltpu.make_async_copy(gq.at[sidx.at[j]], buf,
                                                  gsems[b]).wait()
                            pltpu.async_copy(buf, acc.at[didx.at[j]],
                                             ssems[b], add=True)
                            pltpu.make_async_copy(buf, acc.at[didx.at[j]],
                                                  ssems[b]).wait()
                            jn = j + NBUF

                            @pl.when(jn < CH_ST)
                            def _prefetch():
                                pltpu.async_copy(gq.at[sidx.at[jn]],
                                                 rows.at[pl.ds(b * 128, 128)],
                                                 gsems[b])

                        return t

                    lax.fori_loop(0, CH_ST // NBUF, loop, 0)

                plsc.subcore_barrier()

                for k in range(7):
                    pltpu.sync_copy(acc.at[pl.ds(r0 + k * 448, 448)],
                                    rows.at[pl.ds(0, 448)])
                    pltpu.sync_copy(rows.at[pl.ds(0, 448)],
                                    oq.at[pl.ds(r0 + k * 448, 448)])


_agg_call = pl.kernel(
    _agg_body,
    out_type=[pltpu.HBM((NPAD, F), jnp.float32)] * NQ,
    mesh=_MESH,
    scratch_types=(
        [pltpu.VMEM((CH_ST, 128), jnp.int32),
         pltpu.VMEM((CH_ST, 128), jnp.int32),
         pltpu.VMEM((512, F), jnp.float32),
         pltpu.VMEM_SHARED((NPAD, F), jnp.float32)]
        + [pltpu.SemaphoreType.DMA] * (2 * NBUF)
    ),
    compiler_params=pltpu.CompilerParams(use_tc_tiling_on_sc=False),
)


def _tc1_body(x_ref, w_ref, d0_ref, d1_ref, *out_refs):
    h = jnp.dot(x_ref[...], w_ref[...], preferred_element_type=jnp.float32)
    deg = d0_ref[...] + d1_ref[...] + 1.0
    dinv = lax.rsqrt(deg)
    g = h * dinv
    for q in range(NQ):
        out_refs[q][...] = g[:, q * F:(q + 1) * F]
    out_refs[NQ][...] = dinv


_tc1 = pl.pallas_call(
    _tc1_body,
    grid=(NPAD // 512,),
    in_specs=[
        pl.BlockSpec((512, IN_CH), lambda i: (i, 0)),
        pl.BlockSpec((IN_CH, HID), lambda i: (0, 0)),
        pl.BlockSpec((512, 1), lambda i: (i, 0)),
        pl.BlockSpec((512, 1), lambda i: (i, 0)),
    ],
    out_specs=[pl.BlockSpec((512, F), lambda i: (i, 0))] * NQ
    + [pl.BlockSpec((512, 1), lambda i: (i, 0))],
    out_shape=[jax.ShapeDtypeStruct((NPAD, F), jnp.float32)] * NQ
    + [jax.ShapeDtypeStruct((NPAD, 1), jnp.float32)],
)


def _tcmid_body(*refs):
    a_refs = refs[:NQ]
    dinv_ref, s_ref, w_ref, b_ref = refs[NQ:NQ + 4]
    out_refs = refs[NQ + 4:]
    dinv = dinv_ref[...]
    h = jnp.concatenate([a[...] for a in a_refs], axis=1) * dinv
    h = jnp.maximum(h + b_ref[...], 0.0)
    z = jnp.dot(h, w_ref[...], preferred_element_type=jnp.float32) * s_ref[...]
    for q in range(NQ):
        out_refs[q][...] = z[:, q * F:(q + 1) * F]


_tcmid = pl.pallas_call(
    _tcmid_body,
    grid=(NPAD // 512,),
    in_specs=[pl.BlockSpec((512, F), lambda i: (i, 0))] * NQ + [
        pl.BlockSpec((512, 1), lambda i: (i, 0)),
        pl.BlockSpec((512, 1), lambda i: (i, 0)),
        pl.BlockSpec((HID, HID), lambda i: (0, 0)),
        pl.BlockSpec((1, HID), lambda i: (0, 0)),
    ],
    out_specs=[pl.BlockSpec((512, F), lambda i: (i, 0))] * NQ,
    out_shape=[jax.ShapeDtypeStruct((NPAD, F), jnp.float32)] * NQ,
)


def _tcfin_body(z_ref, bc_ref, o_ref):
    o_ref[...] = jax.nn.sigmoid(z_ref[:, 0:1] + bc_ref[...])


_tcfin = pl.pallas_call(
    _tcfin_body,
    grid=(NPAD // 512,),
    in_specs=[pl.BlockSpec((512, F), lambda i: (i, 0)),
              pl.BlockSpec((1, 1), lambda i: (0, 0))],
    out_specs=pl.BlockSpec((512, 1), lambda i: (i, 0)),
    out_shape=jax.ShapeDtypeStruct((N, 1), jnp.float32),
)


def kernel(x, edge_index, W1, b1, W2, b2, Wc, bc):
    pad = (N + (jnp.arange(EPAD - E, dtype=jnp.int32) % (NPAD - N)))
    srcR = jnp.concatenate([edge_index[0], pad]).reshape(EROWS, 128)
    dstR = jnp.concatenate([edge_index[1], pad]).reshape(EROWS, 128)

    _, d0, d1 = _deg_call(dstR)

    g = _tc1(x, W1, d0.reshape(NPAD, 1), d1.reshape(NPAD, 1))
    dinv = g[NQ]

    Wstack = jnp.stack([W2, jnp.pad(Wc, ((0, 0), (0, HID - 1)))])
    bstack = jnp.stack([b1.reshape(1, HID), b2.reshape(1, HID)])
    sstack = jnp.stack([dinv, jnp.ones_like(dinv)])

    def step(gq, xs):
        Wi, bi, si = xs
        a = _agg_call(srcR, dstR, *gq)
        z = _tcmid(*a, dinv, si, Wi, bi)
        return tuple(z), 0.0

    zq, _ = lax.scan(step, tuple(g[:NQ]), (Wstack, bstack, sstack))
    return _tcfin(zq[0], bc.reshape(1, 1))


# speedup vs baseline: 21.9293x; 1.1762x over previous
import jax
import jax.numpy as jnp
from jax import lax
from jax.experimental import pallas as pl
from jax.experimental.pallas import tpu as pltpu
from jax.experimental.pallas import tpu_sc as plsc

N = 50000
E = 800000
IN_CH = 128
HID = 64
F = 16
NQ = HID // F

NPAD = 50176
EPAD = 819200
EROWS = EPAD // 128
CH_DEG = 200
CH_AGG = 400
CH_ST = 200
TSLICE = NPAD // 16
NBUF = 4
GROWS = NPAD * NQ
GLEN = GROWS - NQ + 1

_MESH = plsc.VectorSubcoreMesh(core_axis_name="c", subcore_axis_name="s")


def _deg_body(dstR, hist_out, out0, out1, idx, hist, rbuf, pbuf):
    c = lax.axis_index("c")
    s = lax.axis_index("s")
    w = c * 16 + s

    def zb(i, t):
        hist[pl.ds(i * 16, 16)] = jnp.zeros((16,), jnp.float32)
        return t

    lax.fori_loop(0, NPAD // 16, zb, 0)
    pltpu.sync_copy(dstR.at[pl.ds(w * CH_DEG, CH_DEG)], idx)

    ones16 = jnp.ones((16,), jnp.float32)

    def loop(j, t):
        def inner(k, u):
            ids = idx[j, pl.ds(k * 16, 16)]
            plsc.addupdate_scatter(hist, [ids], ones16)
            return u

        lax.fori_loop(0, 8, inner, 0)
        return t

    lax.fori_loop(0, CH_DEG, loop, 0)

    pltpu.sync_copy(hist, hist_out.at[w])
    plsc.subcore_barrier()

    r0 = s * TSLICE

    def zr(i, t):
        rbuf[pl.ds(i * 16, 16)] = jnp.zeros((16,), jnp.float32)
        return t

    lax.fori_loop(0, TSLICE // 16, zr, 0)
    for t in range(16):
        pltpu.sync_copy(hist_out.at[c * 16 + t, pl.ds(r0, TSLICE)], pbuf)

        def racc(i, u):
            rbuf[pl.ds(i * 16, 16)] = rbuf[pl.ds(i * 16, 16)] + pbuf[pl.ds(i * 16, 16)]
            return u

        lax.fori_loop(0, TSLICE // 16, racc, 0)
    for cid, oref in ((0, out0), (1, out1)):
        @pl.when(c == cid)
        def _wb(oref=oref):
            pltpu.sync_copy(rbuf, oref.at[pl.ds(r0, TSLICE)])


_deg_call = pl.kernel(
    _deg_body,
    out_type=[pltpu.HBM((32, NPAD), jnp.float32),
              pltpu.HBM((NPAD,), jnp.float32),
              pltpu.HBM((NPAD,), jnp.float32)],
    mesh=_MESH,
    scratch_types=(
        [pltpu.VMEM((CH_DEG, 128), jnp.int32),
         pltpu.VMEM((NPAD,), jnp.float32),
         pltpu.VMEM((TSLICE,), jnp.float32),
         pltpu.VMEM((TSLICE,), jnp.float32)]
    ),
    compiler_params=pltpu.CompilerParams(use_tc_tiling_on_sc=False,
                                         needs_layout_passes=False),
)


def _agg_body(src4R, dstR, gp, out, sidx, didx, rows, acc,
              gs0, gs1, gs2, gs3, as0, as1, as2, as3):
    c = lax.axis_index("c")
    s = lax.axis_index("s")
    gsems = (gs0, gs1, gs2, gs3)
    ssems = (as0, as1, as2, as3)
    r0 = s * TSLICE

    def zb(i, t):
        rows[i, :] = jnp.zeros((16,), jnp.float32)
        return t

    lax.fori_loop(0, 448, zb, 0)

    for cid in (0, 1):
        @pl.when(c == cid)
        def _core(cid=cid):
            for p in range(NQ // 2):
                q = (NQ // 2) * cid + p
                gq = gp.at[pl.ds(q, GLEN)]
                for k in range(7):
                    pltpu.sync_copy(rows.at[pl.ds(0, 448)],
                                    acc.at[pl.ds(r0 + k * 448, 448)])
                plsc.subcore_barrier()

                for st in range(CH_AGG // CH_ST):
                    e0 = s * CH_AGG + st * CH_ST
                    pltpu.sync_copy(src4R.at[pl.ds(e0, CH_ST)], sidx)
                    pltpu.sync_copy(dstR.at[pl.ds(e0, CH_ST)], didx)

                    for b in range(NBUF):
                        pltpu.async_copy(gq.at[sidx.at[b]],
                                         rows.at[pl.ds(b * 128, 128)],
                                         gsems[b])

                    def loop(o, t, gq=gq):
                        for b in range(NBUF):
                            j = o * NBUF + b
                            buf = rows.at[pl.ds(b * 128, 128)]
                            pltpu.make_async_copy(gq.at[sidx.at[j]], buf,
                                                  gsems[b]).wait()
                            pltpu.async_copy(buf, acc.at[didx.at[j]],
                                             ssems[b], add=True)
                            pltpu.make_async_copy(buf, acc.at[didx.at[j]],
                                                  ssems[b]).wait()
                            jn = j + NBUF

                            @pl.when(jn < CH_ST)
                            def _prefetch():
                                pltpu.async_copy(gq.at[sidx.at[jn]],
                                                 rows.at[pl.ds(b * 128, 128)],
                                                 gsems[b])

                        return t

                    lax.fori_loop(0, CH_ST // NBUF, loop, 0)

                plsc.subcore_barrier()

                for k in range(7):
                    pltpu.sync_copy(acc.at[pl.ds(r0 + k * 448, 448)],
                                    rows.at[pl.ds(448, 448)])
                    pltpu.sync_copy(rows.at[pl.ds(448, 448)],
                                    out.at[q, pl.ds(r0 + k * 448, 448)])

                def zb2(i, t):
                    rows[i, :] = jnp.zeros((16,), jnp.float32)
                    return t

                lax.fori_loop(0, 448, zb2, 0)


_agg_call = pl.kernel(
    _agg_body,
    out_type=pltpu.HBM((NQ, NPAD, F), jnp.float32),
    mesh=_MESH,
    scratch_types=(
        [pltpu.VMEM((CH_ST, 128), jnp.int32),
         pltpu.VMEM((CH_ST, 128), jnp.int32),
         pltpu.VMEM((896, F), jnp.float32),
         pltpu.VMEM_SHARED((NPAD, F), jnp.float32)]
        + [pltpu.SemaphoreType.DMA] * (2 * NBUF)
    ),
    compiler_params=pltpu.CompilerParams(use_tc_tiling_on_sc=False),
)


def _tc1_body(x_ref, w_ref, d0_ref, d1_ref, gp_ref, dinv_ref):
    h = jnp.dot(x_ref[...], w_ref[...], preferred_element_type=jnp.float32)
    deg = d0_ref[...] + d1_ref[...] + 1.0
    dinv = lax.rsqrt(deg)
    g = h * dinv
    gp_ref[:, :HID] = g[:256, :]
    gp_ref[:, HID:] = g[256:, :]
    dinv_ref[...] = dinv


_tc1 = pl.pallas_call(
    _tc1_body,
    grid=(NPAD // 512,),
    in_specs=[
        pl.BlockSpec((512, IN_CH), lambda i: (i, 0)),
        pl.BlockSpec((IN_CH, HID), lambda i: (0, 0)),
        pl.BlockSpec((512, 1), lambda i: (i, 0)),
        pl.BlockSpec((512, 1), lambda i: (i, 0)),
    ],
    out_specs=[pl.BlockSpec((256, 128), lambda i: (i, 0)),
               pl.BlockSpec((512, 1), lambda i: (i, 0))],
    out_shape=[jax.ShapeDtypeStruct((NPAD // 2, 128), jnp.float32),
               jax.ShapeDtypeStruct((NPAD, 1), jnp.float32)],
)


def _tcmid_body(a_ref, gp_ref, dinv_ref, s_ref, w_ref, b_ref, z_ref):
    dinv = dinv_ref[...]
    gpv = gp_ref[...]
    g = jnp.concatenate([gpv[:, :HID], gpv[:, HID:]], axis=0)
    agg = jnp.concatenate([a_ref[q] for q in range(NQ)], axis=1)
    h = jnp.maximum((agg + g) * dinv + b_ref[...], 0.0)
    z = jnp.dot(h, w_ref[...], preferred_element_type=jnp.float32) * s_ref[...]
    z_ref[:, :HID] = z[:256, :]
    z_ref[:, HID:] = z[256:, :]


_tcmid = pl.pallas_call(
    _tcmid_body,
    grid=(NPAD // 512,),
    in_specs=[
        pl.BlockSpec((NQ, 512, F), lambda i: (0, i, 0)),
        pl.BlockSpec((256, 128), lambda i: (i, 0)),
        pl.BlockSpec((512, 1), lambda i: (i, 0)),
        pl.BlockSpec((512, 1), lambda i: (i, 0)),
        pl.BlockSpec((HID, HID), lambda i: (0, 0)),
        pl.BlockSpec((1, HID), lambda i: (0, 0)),
    ],
    out_specs=pl.BlockSpec((256, 128), lambda i: (i, 0)),
    out_shape=jax.ShapeDtypeStruct((NPAD // 2, 128), jnp.float32),
)


def _tcfin_body(z_ref, bc_ref, o_ref):
    zp = z_ref[...]
    z0 = jnp.concatenate([zp[:, 0:1], zp[:, HID:HID + 1]], axis=0)
    o_ref[...] = jax.nn.sigmoid(z0 + bc_ref[...])


_tcfin = pl.pallas_call(
    _tcfin_body,
    grid=(NPAD // 512,),
    in_specs=[pl.BlockSpec((256, 128), lambda i: (i, 0)),
              pl.BlockSpec((1, 1), lambda i: (0, 0))],
    out_specs=pl.BlockSpec((512, 1), lambda i: (i, 0)),
    out_shape=jax.ShapeDtypeStruct((N, 1), jnp.float32),
)


def kernel(x, edge_index, W1, b1, W2, b2, Wc, bc):
    pad = (N + (jnp.arange(EPAD - E, dtype=jnp.int32) % (NPAD - N)))
    src = jnp.concatenate([edge_index[0], pad])
    srow = ((src >> 9) * 256 + (src & 255)) * 8 + ((src >> 8) & 1) * 4
    src4R = srow.reshape(EROWS, 128)
    dstR = jnp.concatenate([edge_index[1], pad]).reshape(EROWS, 128)

    _, d0, d1 = _deg_call(dstR)

    gp, dinv = _tc1(x, W1, d0.reshape(NPAD, 1), d1.reshape(NPAD, 1))

    Wstack = jnp.stack([W2, jnp.pad(Wc, ((0, 0), (0, HID - 1)))])
    bstack = jnp.stack([b1.reshape(1, HID), b2.reshape(1, HID)])
    sstack = jnp.stack([dinv, jnp.ones_like(dinv)])

    def step(g_packed, xs):
        Wi, bi, si = xs
        a = _agg_call(src4R, dstR, g_packed.reshape(GROWS, F))
        z = _tcmid(a, g_packed, dinv, si, Wi, bi)
        return z, 0.0

    zp, _ = lax.scan(step, gp, (Wstack, bstack, sstack))
    return _tcfin(zp, bc.reshape(1, 1))


# speedup vs baseline: 26.1249x; 1.1913x over previous
import jax
import jax.numpy as jnp
from jax import lax
from jax.experimental import pallas as pl
from jax.experimental.pallas import tpu as pltpu
from jax.experimental.pallas import tpu_sc as plsc

N = 50000
E = 800000
IN_CH = 128
HID = 64
F = 32
NQ = HID // F

NPAD = 50176
EPAD = 819200
EROWS = EPAD // 128
CH_DEG = 200
CH_AGG = 400
CH_ST = 20
TSLICE = NPAD // 16
NBUF = 4
GROWS = NPAD * NQ
GLEN = GROWS - NQ + 1

_MESH = plsc.VectorSubcoreMesh(core_axis_name="c", subcore_axis_name="s")


def _deg_body(dstR, hist_out, out0, out1, idx, hist, rbuf, pbuf):
    c = lax.axis_index("c")
    s = lax.axis_index("s")
    w = c * 16 + s

    def zb(i, t):
        hist[pl.ds(i * 16, 16)] = jnp.zeros((16,), jnp.float32)
        return t

    lax.fori_loop(0, NPAD // 16, zb, 0)
    pltpu.sync_copy(dstR.at[pl.ds(w * CH_DEG, CH_DEG)], idx)

    ones16 = jnp.ones((16,), jnp.float32)

    def loop(j, t):
        def inner(k, u):
            ids = idx[j, pl.ds(k * 16, 16)]
            plsc.addupdate_scatter(hist, [ids], ones16)
            return u

        lax.fori_loop(0, 8, inner, 0)
        return t

    lax.fori_loop(0, CH_DEG, loop, 0)

    pltpu.sync_copy(hist, hist_out.at[w])
    plsc.subcore_barrier()

    r0 = s * TSLICE

    def zr(i, t):
        rbuf[pl.ds(i * 16, 16)] = jnp.zeros((16,), jnp.float32)
        return t

    lax.fori_loop(0, TSLICE // 16, zr, 0)
    for t in range(16):
        pltpu.sync_copy(hist_out.at[c * 16 + t, pl.ds(r0, TSLICE)], pbuf)

        def racc(i, u):
            rbuf[pl.ds(i * 16, 16)] = rbuf[pl.ds(i * 16, 16)] + pbuf[pl.ds(i * 16, 16)]
            return u

        lax.fori_loop(0, TSLICE // 16, racc, 0)
    for cid, oref in ((0, out0), (1, out1)):
        @pl.when(c == cid)
        def _wb(oref=oref):
            pltpu.sync_copy(rbuf, oref.at[pl.ds(r0, TSLICE)])


_deg_call = pl.kernel(
    _deg_body,
    out_type=[pltpu.HBM((32, NPAD), jnp.float32),
              pltpu.HBM((NPAD,), jnp.float32),
              pltpu.HBM((NPAD,), jnp.float32)],
    mesh=_MESH,
    scratch_types=(
        [pltpu.VMEM((CH_DEG, 128), jnp.int32),
         pltpu.VMEM((NPAD,), jnp.float32),
         pltpu.VMEM((TSLICE,), jnp.float32),
         pltpu.VMEM((TSLICE,), jnp.float32)]
    ),
    compiler_params=pltpu.CompilerParams(use_tc_tiling_on_sc=False,
                                         needs_layout_passes=False),
)


def _agg_body(src4R, dstR, gp, out, sidx, didx, ebuf, zwbuf, acc,
              gs0, gs1, gs2, gs3, as0, as1, as2, as3):
    c = lax.axis_index("c")
    s = lax.axis_index("s")
    gsems = (gs0, gs1, gs2, gs3)
    ssems = (as0, as1, as2, as3)
    r0 = s * TSLICE

    def zb(i, t):
        zwbuf[i, pl.ds(0, 16)] = jnp.zeros((16,), jnp.float32)
        zwbuf[i, pl.ds(16, 16)] = jnp.zeros((16,), jnp.float32)
        return t

    lax.fori_loop(0, 112, zb, 0)

    for cid in (0, 1):
        @pl.when(c == cid)
        def _core(cid=cid):
            q = cid
            gq = gp.at[pl.ds(q, GLEN)]
            for k in range(28):
                pltpu.sync_copy(zwbuf, acc.at[pl.ds(r0 + k * 112, 112)])
            plsc.subcore_barrier()

            for st in range(CH_AGG // CH_ST):
                e0 = s * CH_AGG + st * CH_ST
                pltpu.sync_copy(src4R.at[pl.ds(e0, CH_ST)], sidx)
                pltpu.sync_copy(dstR.at[pl.ds(e0, CH_ST)], didx)

                for b in range(NBUF):
                    pltpu.async_copy(gq.at[sidx.at[b]],
                                     ebuf.at[pl.ds(b * 128, 128)],
                                     gsems[b])

                def loop(o, t, gq=gq):
                    for b in range(NBUF):
                        j = o * NBUF + b
                        buf = ebuf.at[pl.ds(b * 128, 128)]
                        pltpu.make_async_copy(gq.at[sidx.at[j]], buf,
                                              gsems[b]).wait()
                        pltpu.async_copy(buf, acc.at[didx.at[j]],
                                         ssems[b], add=True)
                        pltpu.make_async_copy(buf, acc.at[didx.at[j]],
                                              ssems[b]).wait()
                        jn = j + NBUF

                        @pl.when(jn < CH_ST)
                        def _prefetch():
                            pltpu.async_copy(gq.at[sidx.at[jn]],
                                             ebuf.at[pl.ds(b * 128, 128)],
                                             gsems[b])

                    return t

                lax.fori_loop(0, CH_ST // NBUF, loop, 0)

            plsc.subcore_barrier()

            for k in range(28):
                pltpu.sync_copy(acc.at[pl.ds(r0 + k * 112, 112)], zwbuf)
                pltpu.sync_copy(zwbuf, out.at[q, pl.ds(r0 + k * 112, 112)])


_agg_call = pl.kernel(
    _agg_body,
    out_type=pltpu.HBM((NQ, NPAD, F), jnp.float32),
    mesh=_MESH,
    scratch_types=(
        [pltpu.VMEM((CH_ST, 128), jnp.int32),
         pltpu.VMEM((CH_ST, 128), jnp.int32),
         pltpu.VMEM((512, F), jnp.float32),
         pltpu.VMEM((112, F), jnp.float32),
         pltpu.VMEM_SHARED((NPAD, F), jnp.float32)]
        + [pltpu.SemaphoreType.DMA] * (2 * NBUF)
    ),
    compiler_params=pltpu.CompilerParams(use_tc_tiling_on_sc=False),
)


def _tc1_body(x_ref, w_ref, d0_ref, d1_ref, gp_ref, dinv_ref):
    h = jnp.dot(x_ref[...], w_ref[...], preferred_element_type=jnp.float32)
    deg = d0_ref[...] + d1_ref[...] + 1.0
    dinv = lax.rsqrt(deg)
    g = h * dinv
    gp_ref[:, :HID] = g[:256, :]
    gp_ref[:, HID:] = g[256:, :]
    dinv_ref[...] = dinv


_tc1 = pl.pallas_call(
    _tc1_body,
    grid=(NPAD // 512,),
    in_specs=[
        pl.BlockSpec((512, IN_CH), lambda i: (i, 0)),
        pl.BlockSpec((IN_CH, HID), lambda i: (0, 0)),
        pl.BlockSpec((512, 1), lambda i: (i, 0)),
        pl.BlockSpec((512, 1), lambda i: (i, 0)),
    ],
    out_specs=[pl.BlockSpec((256, 128), lambda i: (i, 0)),
               pl.BlockSpec((512, 1), lambda i: (i, 0))],
    out_shape=[jax.ShapeDtypeStruct((NPAD // 2, 128), jnp.float32),
               jax.ShapeDtypeStruct((NPAD, 1), jnp.float32)],
)


def _tcmid_body(a_ref, gp_ref, dinv_ref, s_ref, w_ref, b_ref, z_ref):
    dinv = dinv_ref[...]
    gpv = gp_ref[...]
    g = jnp.concatenate([gpv[:, :HID], gpv[:, HID:]], axis=0)
    agg = jnp.concatenate([a_ref[q] for q in range(NQ)], axis=1)
    h = jnp.maximum((agg + g) * dinv + b_ref[...], 0.0)
    z = jnp.dot(h, w_ref[...], preferred_element_type=jnp.float32) * s_ref[...]
    z_ref[:, :HID] = z[:256, :]
    z_ref[:, HID:] = z[256:, :]


_tcmid = pl.pallas_call(
    _tcmid_body,
    grid=(NPAD // 512,),
    in_specs=[
        pl.BlockSpec((NQ, 512, F), lambda i: (0, i, 0)),
        pl.BlockSpec((256, 128), lambda i: (i, 0)),
        pl.BlockSpec((512, 1), lambda i: (i, 0)),
        pl.BlockSpec((512, 1), lambda i: (i, 0)),
        pl.BlockSpec((HID, HID), lambda i: (0, 0)),
        pl.BlockSpec((1, HID), lambda i: (0, 0)),
    ],
    out_specs=pl.BlockSpec((256, 128), lambda i: (i, 0)),
    out_shape=jax.ShapeDtypeStruct((NPAD // 2, 128), jnp.float32),
)


def _tcfin_body(z_ref, bc_ref, o_ref):
    zp = z_ref[...]
    z0 = jnp.concatenate([zp[:, 0:1], zp[:, HID:HID + 1]], axis=0)
    o_ref[...] = jax.nn.sigmoid(z0 + bc_ref[...])


_tcfin = pl.pallas_call(
    _tcfin_body,
    grid=(NPAD // 512,),
    in_specs=[pl.BlockSpec((256, 128), lambda i: (i, 0)),
              pl.BlockSpec((1, 1), lambda i: (0, 0))],
    out_specs=pl.BlockSpec((512, 1), lambda i: (i, 0)),
    out_shape=jax.ShapeDtypeStruct((N, 1), jnp.float32),
)


def kernel(x, edge_index, W1, b1, W2, b2, Wc, bc):
    pad = (N + (jnp.arange(EPAD - E, dtype=jnp.int32) % (NPAD - N)))
    src = jnp.concatenate([edge_index[0], pad])
    srow = ((src >> 9) * 256 + (src & 255)) * 4 + ((src >> 8) & 1) * 2
    src4R = srow.reshape(EROWS, 128)
    dstR = jnp.concatenate([edge_index[1], pad]).reshape(EROWS, 128)

    _, d0, d1 = _deg_call(dstR)

    gp, dinv = _tc1(x, W1, d0.reshape(NPAD, 1), d1.reshape(NPAD, 1))

    Wstack = jnp.stack([W2, jnp.pad(Wc, ((0, 0), (0, HID - 1)))])
    bstack = jnp.stack([b1.reshape(1, HID), b2.reshape(1, HID)])
    sstack = jnp.stack([dinv, jnp.ones_like(dinv)])

    def step(g_packed, xs):
        Wi, bi, si = xs
        a = _agg_call(src4R, dstR, g_packed.reshape(GROWS, F))
        z = _tcmid(a, g_packed, dinv, si, Wi, bi)
        return z, 0.0

    zp, _ = lax.scan(step, gp, (Wstack, bstack, sstack))
    return _tcfin(zp, bc.reshape(1, 1))
